# baseline (speedup 1.0000x reference)
import functools

import jax
import jax.numpy as jnp
from jax import lax
from jax.experimental import pallas as pl
from jax.experimental.pallas import tpu as pltpu
from jax.experimental.pallas import tpu_sc as plsc

_LANES = 128


def _cbpost_body(cb_ref, wpost_t_ref, bpost_ref, cbp_ref, nrm2_ref):
    cbb = cb_ref[...]
    cbp_ref[...] = jnp.dot(cbb, wpost_t_ref[...],
                           preferred_element_type=jnp.float32) + bpost_ref[...]
    nrm2_ref[...] = jnp.sum(cbb * cbb, axis=1).reshape(nrm2_ref.shape)


def _cbpost(codebook, wpost_t, bpost, block_rows):
    c, e = codebook.shape
    d = wpost_t.shape[1]
    nblk = c // block_rows
    cbp, nrm2 = pl.pallas_call(
        _cbpost_body,
        grid=(nblk,),
        in_specs=[
            pl.BlockSpec((block_rows, e), lambda i: (i, 0)),
            pl.BlockSpec((e, d), lambda i: (0, 0)),
            pl.BlockSpec((1, d), lambda i: (0, 0)),
        ],
        out_specs=[
            pl.BlockSpec((block_rows, d), lambda i: (i, 0)),
            pl.BlockSpec((1, 1, block_rows), lambda i: (i, 0, 0)),
        ],
        out_shape=[
            jax.ShapeDtypeStruct((c, d), jnp.float32),
            jax.ShapeDtypeStruct((nblk, 1, block_rows), jnp.float32),
        ],
    )(codebook, wpost_t, bpost)
    return cbp, nrm2.reshape(1, c)


def _assign_body(z_ref, wpre_t_ref, bpre_ref, cbt_ref, nrm2_ref,
                 tok_ref, loss_ref):
    i = pl.program_id(0)
    zb = z_ref[...]
    zp = jnp.dot(zb, wpre_t_ref[...],
                 preferred_element_type=jnp.float32) + bpre_ref[...]
    norm = jnp.sqrt(jnp.sum(zp * zp, axis=1, keepdims=True))
    zn = zp / jnp.maximum(norm, 1e-12)
    zsq = jnp.sum(zn * zn, axis=1)
    s = jnp.dot(zn, cbt_ref[...],
                preferred_element_type=jnp.float32)
    r, c = s.shape
    g_cnt = c // _LANES

    m_run = s[:, 0:_LANES]
    g_run = jnp.zeros((r, _LANES), jnp.int32)
    n_run = jnp.broadcast_to(nrm2_ref[:, 0:_LANES], (r, _LANES))
    for g in range(1, g_cnt):
        sg = s[:, g * _LANES:(g + 1) * _LANES]
        ng = jnp.broadcast_to(nrm2_ref[:, g * _LANES:(g + 1) * _LANES],
                              (r, _LANES))
        gt = sg > m_run
        m_run = jnp.where(gt, sg, m_run)
        g_run = jnp.where(gt, g, g_run)
        n_run = jnp.where(gt, ng, n_run)

    lane = lax.broadcasted_iota(jnp.int32, (r, _LANES), 1)
    full_idx = g_run * _LANES + lane
    maxv = jnp.max(m_run, axis=1, keepdims=True)
    eq = m_run == maxv
    tok = jnp.min(jnp.where(eq, full_idx, jnp.int32(0x7FFFFFFF)), axis=1)
    sel = full_idx == tok[:, None]
    nsel = jnp.sum(jnp.where(sel, n_run, 0.0), axis=1)
    part = jnp.sum(zsq + nsel) - 2.0 * jnp.sum(maxv)

    tok_ref[...] = tok.reshape(tok_ref.shape)

    @pl.when(i == 0)
    def _():
        loss_ref[...] = jnp.zeros(loss_ref.shape, loss_ref.dtype)

    loss_ref[...] = loss_ref[...] + part


def _assign(z2, wpre_t, bpre, cbt, nrm2, block_rows):
    n, d = z2.shape
    e = wpre_t.shape[1]
    c = cbt.shape[1]
    nblk = n // block_rows
    tok3, loss_sum = pl.pallas_call(
        _assign_body,
        grid=(nblk,),
        in_specs=[
            pl.BlockSpec((block_rows, d), lambda i: (i, 0)),
            pl.BlockSpec((d, e), lambda i: (0, 0)),
            pl.BlockSpec((1, e), lambda i: (0, 0)),
            pl.BlockSpec((e, c), lambda i: (0, 0)),
            pl.BlockSpec((1, c), lambda i: (0, 0)),
        ],
        out_specs=[
            pl.BlockSpec((1, 1, block_rows), lambda i: (i, 0, 0)),
            pl.BlockSpec((1, 1), lambda i: (0, 0)),
        ],
        out_shape=[
            jax.ShapeDtypeStruct((nblk, 1, block_rows), jnp.int32),
            jax.ShapeDtypeStruct((1, 1), jnp.float32),
        ],
    )(z2, wpre_t, bpre, cbt, nrm2)
    return tok3.reshape(n), loss_sum


def _make_gather(d, n):
    info = plsc.get_sparse_core_info()
    nw = info.num_cores * info.num_subcores
    b_per_w = n // nw
    ch = 48
    nbuf = 3
    nch = b_per_w // ch
    mesh = plsc.VectorSubcoreMesh(core_axis_name="c", subcore_axis_name="s")

    @functools.partial(
        pl.kernel, mesh=mesh,
        out_type=jax.ShapeDtypeStruct((n, d), jnp.float32),
        scratch_types=[
            pltpu.VMEM((b_per_w,), jnp.int32),
            pltpu.VMEM((nbuf, ch, d), jnp.float32),
            pltpu.SemaphoreType.DMA,
            pltpu.SemaphoreType.DMA,
        ],
    )
    def gk(cbp_hbm, idx_hbm, out_hbm, idx_v, bufs_v, sem_g, sem_o):
        wid = lax.axis_index("s") * info.num_cores + lax.axis_index("c")
        base = wid * b_per_w

        def gather(j):
            return pltpu.async_copy(cbp_hbm.at[idx_v.at[pl.ds(j * ch, ch)]],
                                    bufs_v.at[j % nbuf], sem_g)

        def put(j):
            return pltpu.async_copy(bufs_v.at[j % nbuf],
                                    out_hbm.at[pl.ds(base + j * ch, ch)],
                                    sem_o)

        pltpu.sync_copy(idx_hbm.at[pl.ds(base, b_per_w)], idx_v)
        g = [None] * nch
        o = [None] * nch
        for j in range(min(nbuf, nch)):
            g[j] = gather(j)
        for j in range(nch):
            if j >= 1:
                o[j - 1].wait()
                if j + nbuf - 1 < nch:
                    g[j + nbuf - 1] = gather(j + nbuf - 1)
            g[j].wait()
            o[j] = put(j)
        o[nch - 1].wait()

    return gk


def kernel(z, W_pre, b_pre, codebook, W_post, b_post):
    B, T, K, D = z.shape
    C, E = codebook.shape
    N = B * T * K

    cbp, nrm2 = _cbpost(codebook, W_post.T, b_post.reshape(1, D),
                        block_rows=512)
    z2 = z.reshape(N, D)
    tokens, loss_sum = _assign(z2, W_pre.T, b_pre.reshape(1, E),
                               codebook.T, nrm2, block_rows=512)
    gk = _make_gather(D, N)
    out2 = gk(cbp, tokens)

    out = out2.reshape(B, T, K, D)
    commitment_loss = loss_sum[0, 0] * (0.02 / (N * E))
    return (out, tokens.reshape(B, T, K), commitment_loss)


# speedup vs baseline: 1.7099x; 1.7099x over previous
import functools

import jax
import jax.numpy as jnp
from jax import lax
from jax.experimental import pallas as pl
from jax.experimental.pallas import tpu as pltpu
from jax.experimental.pallas import tpu_sc as plsc

_LANES = 128
_CONTRACT_MINOR = (((1,), (1,)), ((), ()))


def _assign_body(z_ref, wpre_ref, bpre_ref, cb_ref, tok_ref, zn_ref):
    zb4 = z_ref[...]
    r = zb4.shape[1] * zb4.shape[2]
    zb = zb4.reshape(r, zb4.shape[3])
    zp = lax.dot_general(zb, wpre_ref[...], _CONTRACT_MINOR,
                         preferred_element_type=jnp.float32) + bpre_ref[...]
    norm = jnp.sqrt(jnp.sum(zp * zp, axis=1, keepdims=True))
    zn = zp / jnp.maximum(norm, 1e-12)
    zn_ref[...] = zn
    s = lax.dot_general(zn, cb_ref[...], _CONTRACT_MINOR,
                        preferred_element_type=jnp.float32)
    c = s.shape[1]
    g_cnt = c // _LANES

    m_run = s[:, 0:_LANES]
    g_run = jnp.zeros((r, _LANES), jnp.int32)
    for g in range(1, g_cnt):
        sg = s[:, g * _LANES:(g + 1) * _LANES]
        gt = sg > m_run
        m_run = jnp.where(gt, sg, m_run)
        g_run = jnp.where(gt, g, g_run)

    lane = lax.broadcasted_iota(jnp.int32, (r, _LANES), 1)
    full_idx = g_run * _LANES + lane
    maxv = jnp.max(m_run, axis=1, keepdims=True)
    eq = m_run == maxv
    tok = jnp.min(jnp.where(eq, full_idx, jnp.int32(0x7FFFFFFF)), axis=1)
    tok_ref[...] = tok.reshape(tok_ref.shape)


def _assign(z4, wpre, bpre, cb, tb):
    b, t, k, d = z4.shape
    e = wpre.shape[0]
    n = b * t * k
    r = tb * k
    nt = t // tb
    tok4, zn = pl.pallas_call(
        _assign_body,
        grid=(b, nt),
        in_specs=[
            pl.BlockSpec((1, tb, k, d), lambda i, j: (i, j, 0, 0)),
            pl.BlockSpec(wpre.shape, lambda i, j: (0, 0)),
            pl.BlockSpec((1, e), lambda i, j: (0, 0)),
            pl.BlockSpec(cb.shape, lambda i, j: (0, 0)),
        ],
        out_specs=[
            pl.BlockSpec((1, 1, 1, r), lambda i, j: (i, j, 0, 0)),
            pl.BlockSpec((r, e), lambda i, j: (i * nt + j, 0)),
        ],
        out_shape=[
            jax.ShapeDtypeStruct((b, nt, 1, r), jnp.int32),
            jax.ShapeDtypeStruct((n, e), jnp.float32),
        ],
    )(z4, wpre, bpre, cb)
    return tok4.reshape(n), zn


def _make_gather(d, n):
    info = plsc.get_sparse_core_info()
    nw = info.num_cores * info.num_subcores
    b_per_w = n // nw
    ch = 128
    nch = b_per_w // ch
    mesh = plsc.VectorSubcoreMesh(core_axis_name="c", subcore_axis_name="s")

    @functools.partial(
        pl.kernel, mesh=mesh,
        out_type=jax.ShapeDtypeStruct((n, d), jnp.float32),
        scratch_types=[
            pltpu.VMEM((b_per_w,), jnp.int32),
            pltpu.VMEM((2, ch, d), jnp.float32),
            pltpu.SemaphoreType.DMA,
            pltpu.SemaphoreType.DMA,
        ],
    )
    def gk(cb_hbm, idx_hbm, q_hbm, idx_v, bufs_v, sem_g, sem_o):
        wid = lax.axis_index("s") * info.num_cores + lax.axis_index("c")
        base = wid * b_per_w

        def gather(j):
            return pltpu.async_copy(cb_hbm.at[idx_v.at[pl.ds(j * ch, ch)]],
                                    bufs_v.at[j % 2], sem_g)

        def put(j):
            return pltpu.async_copy(bufs_v.at[j % 2],
                                    q_hbm.at[pl.ds(base + j * ch, ch)],
                                    sem_o)

        pltpu.sync_copy(idx_hbm.at[pl.ds(base, b_per_w)], idx_v)
        g = [None] * nch
        o = [None] * nch
        g[0] = gather(0)
        g[1] = gather(1)
        for j in range(nch):
            if j >= 1:
                o[j - 1].wait()
                if j + 1 < nch:
                    g[j + 1] = gather(j + 1)
            g[j].wait()
            o[j] = put(j)
        o[nch - 1].wait()

    return gk


def _out_body(q_ref, zn_ref, wpost_ref, bpost_ref, out_ref, loss_ref):
    i = pl.program_id(0)
    j = pl.program_id(1)
    e = zn_ref.shape[1]
    q = q_ref[...][:, :e]
    res = lax.dot_general(q, wpost_ref[...], _CONTRACT_MINOR,
                          preferred_element_type=jnp.float32) + bpost_ref[...]
    out_ref[...] = res.reshape(out_ref.shape)
    dd = zn_ref[...] - q

    @pl.when(jnp.logical_and(i == 0, j == 0))
    def _():
        loss_ref[...] = jnp.zeros(loss_ref.shape, loss_ref.dtype)

    loss_ref[...] = loss_ref[...] + jnp.sum(dd * dd)


def _project_out(q, zn, wpost, bpost, b, t, k, tb):
    n, ep = q.shape
    e = zn.shape[1]
    d = wpost.shape[0]
    r = tb * k
    nt = t // tb
    out, loss_sum = pl.pallas_call(
        _out_body,
        grid=(b, nt),
        in_specs=[
            pl.BlockSpec((r, ep), lambda i, j: (i * nt + j, 0)),
            pl.BlockSpec((r, e), lambda i, j: (i * nt + j, 0)),
            pl.BlockSpec(wpost.shape, lambda i, j: (0, 0)),
            pl.BlockSpec((1, d), lambda i, j: (0, 0)),
        ],
        out_specs=[
            pl.BlockSpec((1, tb, k, d), lambda i, j: (i, j, 0, 0)),
            pl.BlockSpec((1, 1), lambda i, j: (0, 0)),
        ],
        out_shape=[
            jax.ShapeDtypeStruct((b, t, k, d), jnp.float32),
            jax.ShapeDtypeStruct((1, 1), jnp.float32),
        ],
    )(q, zn, wpost, bpost)
    return out, loss_sum


def kernel(z, W_pre, b_pre, codebook, W_post, b_post):
    B, T, K, D = z.shape
    C, E = codebook.shape
    N = B * T * K
    TB = 144

    tokens, zn = _assign(z, W_pre, b_pre.reshape(1, E), codebook, tb=TB)

    cb_pad = jnp.pad(codebook, ((0, 0), (0, _LANES - E)))
    gk = _make_gather(_LANES, N)
    q = gk(cb_pad, tokens)

    out, loss_sum = _project_out(q, zn, W_post, b_post.reshape(1, D),
                                 B, T, K, tb=TB)
    commitment_loss = loss_sum[0, 0] * (0.02 / (N * E))
    return (out, tokens.reshape(B, T, K), commitment_loss)


# speedup vs baseline: 1.8776x; 1.0980x over previous
import functools

import jax
import jax.numpy as jnp
from jax import lax
from jax.experimental import pallas as pl
from jax.experimental.pallas import tpu as pltpu
from jax.experimental.pallas import tpu_sc as plsc

_LANES = 128
_CONTRACT_MINOR = (((1,), (1,)), ((), ()))


def _assign_body(z_ref, wpre_ref, bpre_ref, cb_ref, tok_ref, zn_ref):
    zb4 = z_ref[...]
    r = zb4.shape[1] * zb4.shape[2]
    zb = zb4.reshape(r, zb4.shape[3])
    zp = lax.dot_general(zb, wpre_ref[...], _CONTRACT_MINOR,
                         preferred_element_type=jnp.float32) + bpre_ref[...]
    norm = jnp.sqrt(jnp.sum(zp * zp, axis=1, keepdims=True))
    zn = zp / jnp.maximum(norm, 1e-12)
    zn_ref[...] = zn
    s = lax.dot_general(zn, cb_ref[...], _CONTRACT_MINOR,
                        preferred_element_type=jnp.float32)
    c = s.shape[1]
    g_cnt = c // _LANES

    m_run = s[:, 0:_LANES]
    g_run = jnp.zeros((r, _LANES), jnp.int32)
    for g in range(1, g_cnt):
        sg = s[:, g * _LANES:(g + 1) * _LANES]
        gt = sg > m_run
        m_run = jnp.where(gt, sg, m_run)
        g_run = jnp.where(gt, g, g_run)

    lane = lax.broadcasted_iota(jnp.int32, (r, _LANES), 1)
    full_idx = g_run * _LANES + lane
    maxv = jnp.max(m_run, axis=1, keepdims=True)
    eq = m_run == maxv
    tok = jnp.min(jnp.where(eq, full_idx, jnp.int32(0x7FFFFFFF)), axis=1)
    tok_ref[...] = tok.reshape(tok_ref.shape)


def _assign(z4, wpre, bpre, cb, tb):
    b, t, k, d = z4.shape
    e = wpre.shape[0]
    n = b * t * k
    r = tb * k
    nt = t // tb
    tok4, zn = pl.pallas_call(
        _assign_body,
        grid=(b, nt),
        in_specs=[
            pl.BlockSpec((1, tb, k, d), lambda i, j: (i, j, 0, 0)),
            pl.BlockSpec(wpre.shape, lambda i, j: (0, 0)),
            pl.BlockSpec((1, e), lambda i, j: (0, 0)),
            pl.BlockSpec(cb.shape, lambda i, j: (0, 0)),
        ],
        out_specs=[
            pl.BlockSpec((1, 1, 1, r), lambda i, j: (i, j, 0, 0)),
            pl.BlockSpec((r, e), lambda i, j: (i * nt + j, 0)),
        ],
        out_shape=[
            jax.ShapeDtypeStruct((b, nt, 1, r), jnp.int32),
            jax.ShapeDtypeStruct((n, e), jnp.float32),
        ],
    )(z4, wpre, bpre, cb)
    return tok4.reshape(n), zn


def _make_gather(d, n):
    info = plsc.get_sparse_core_info()
    nw = info.num_cores * info.num_subcores
    b_per_w = n // nw
    ch = 128
    nch = b_per_w // ch
    mesh = plsc.VectorSubcoreMesh(core_axis_name="c", subcore_axis_name="s")

    @functools.partial(
        pl.kernel, mesh=mesh,
        out_type=jax.ShapeDtypeStruct((n, d), jnp.float32),
        scratch_types=[
            pltpu.VMEM((b_per_w,), jnp.int32),
            pltpu.VMEM((2, ch, d), jnp.float32),
            pltpu.SemaphoreType.DMA,
            pltpu.SemaphoreType.DMA,
        ],
    )
    def gk(cb_hbm, idx_hbm, q_hbm, idx_v, bufs_v, sem_g, sem_o):
        wid = lax.axis_index("s") * info.num_cores + lax.axis_index("c")
        base = wid * b_per_w

        def gather(j):
            return pltpu.async_copy(cb_hbm.at[idx_v.at[pl.ds(j * ch, ch)]],
                                    bufs_v.at[j % 2], sem_g)

        def put(j):
            return pltpu.async_copy(bufs_v.at[j % 2],
                                    q_hbm.at[pl.ds(base + j * ch, ch)],
                                    sem_o)

        pltpu.sync_copy(idx_hbm.at[pl.ds(base, b_per_w)], idx_v)
        g = [None] * nch
        o = [None] * nch
        g[0] = gather(0)
        g[1] = gather(1)
        for j in range(nch):
            if j >= 1:
                o[j - 1].wait()
                if j + 1 < nch:
                    g[j + 1] = gather(j + 1)
            g[j].wait()
            o[j] = put(j)
        o[nch - 1].wait()

    return gk


def _out_body(q_ref, zn_ref, wpost_ref, bpost_ref, out_ref, loss_ref):
    i = pl.program_id(0)
    j = pl.program_id(1)
    e = zn_ref.shape[1]
    q = q_ref[...][:, :e]
    res = lax.dot_general(q, wpost_ref[...], _CONTRACT_MINOR,
                          preferred_element_type=jnp.float32) + bpost_ref[...]
    out_ref[...] = res.reshape(out_ref.shape)
    dd = zn_ref[...] - q

    @pl.when(jnp.logical_and(i == 0, j == 0))
    def _():
        loss_ref[...] = jnp.zeros(loss_ref.shape, loss_ref.dtype)

    loss_ref[...] = loss_ref[...] + jnp.sum(dd * dd)


def _project_out(q, zn, wpost, bpost, b, t, k, tb):
    n, ep = q.shape
    e = zn.shape[1]
    d = wpost.shape[0]
    r = tb * k
    nt = t // tb
    out, loss_sum = pl.pallas_call(
        _out_body,
        grid=(b, nt),
        in_specs=[
            pl.BlockSpec((r, ep), lambda i, j: (i * nt + j, 0)),
            pl.BlockSpec((r, e), lambda i, j: (i * nt + j, 0)),
            pl.BlockSpec(wpost.shape, lambda i, j: (0, 0)),
            pl.BlockSpec((1, d), lambda i, j: (0, 0)),
        ],
        out_specs=[
            pl.BlockSpec((1, tb, k, d), lambda i, j: (i, j, 0, 0)),
            pl.BlockSpec((1, 1), lambda i, j: (0, 0)),
        ],
        out_shape=[
            jax.ShapeDtypeStruct((b, t, k, d), jnp.float32),
            jax.ShapeDtypeStruct((1, 1), jnp.float32),
        ],
    )(q, zn, wpost, bpost)
    return out, loss_sum


def kernel(z, W_pre, b_pre, codebook, W_post, b_post):
    B, T, K, D = z.shape
    C, E = codebook.shape
    N = B * T * K
    TB = 288

    tokens, zn = _assign(z, W_pre, b_pre.reshape(1, E), codebook, tb=TB)

    cb_pad = jnp.pad(codebook, ((0, 0), (0, _LANES - E)))
    gk = _make_gather(_LANES, N)
    q = gk(cb_pad, tokens)

    out, loss_sum = _project_out(q, zn, W_post, b_post.reshape(1, D),
                                 B, T, K, tb=TB)
    commitment_loss = loss_sum[0, 0] * (0.02 / (N * E))
    return (out, tokens.reshape(B, T, K), commitment_loss)


# speedup vs baseline: 1.9140x; 1.0194x over previous
import functools

import jax
import jax.numpy as jnp
from jax import lax
from jax.experimental import pallas as pl
from jax.experimental.pallas import tpu as pltpu
from jax.experimental.pallas import tpu_sc as plsc

_LANES = 128
_CONTRACT_MINOR = (((1,), (1,)), ((), ()))


def _assign_body(z_ref, wpre_ref, bpre_ref, cb_ref, tok_ref, zn_ref):
    zb4 = z_ref[...]
    r = zb4.shape[1] * zb4.shape[2]
    zb = zb4.reshape(r, zb4.shape[3])
    zp = lax.dot_general(zb, wpre_ref[...], _CONTRACT_MINOR,
                         preferred_element_type=jnp.float32) + bpre_ref[...]
    norm = jnp.sqrt(jnp.sum(zp * zp, axis=1, keepdims=True))
    zn = zp / jnp.maximum(norm, 1e-12)
    zn_ref[...] = zn
    s = lax.dot_general(zn, cb_ref[...], _CONTRACT_MINOR,
                        preferred_element_type=jnp.float32)
    c = s.shape[1]
    g_cnt = c // _LANES

    m_run = s[:, 0:_LANES]
    g_run = jnp.zeros((r, _LANES), jnp.int32)
    for g in range(1, g_cnt):
        sg = s[:, g * _LANES:(g + 1) * _LANES]
        gt = sg > m_run
        m_run = jnp.where(gt, sg, m_run)
        g_run = jnp.where(gt, g, g_run)

    lane = lax.broadcasted_iota(jnp.int32, (r, _LANES), 1)
    full_idx = g_run * _LANES + lane
    maxv = jnp.max(m_run, axis=1, keepdims=True)
    eq = m_run == maxv
    tok = jnp.min(jnp.where(eq, full_idx, jnp.int32(0x7FFFFFFF)), axis=1)
    tok_ref[...] = tok.reshape(tok_ref.shape)


def _assign(z4, wpre, bpre, cb, tb):
    b, t, k, d = z4.shape
    e = wpre.shape[0]
    n = b * t * k
    r = tb * k
    nt = t // tb
    tok4, zn = pl.pallas_call(
        _assign_body,
        grid=(b, nt),
        in_specs=[
            pl.BlockSpec((1, tb, k, d), lambda i, j: (i, j, 0, 0)),
            pl.BlockSpec(wpre.shape, lambda i, j: (0, 0)),
            pl.BlockSpec((1, e), lambda i, j: (0, 0)),
            pl.BlockSpec(cb.shape, lambda i, j: (0, 0)),
        ],
        out_specs=[
            pl.BlockSpec((1, 1, 1, r), lambda i, j: (i, j, 0, 0)),
            pl.BlockSpec((r, e), lambda i, j: (i * nt + j, 0)),
        ],
        out_shape=[
            jax.ShapeDtypeStruct((b, nt, 1, r), jnp.int32),
            jax.ShapeDtypeStruct((n, e), jnp.float32),
        ],
    )(z4, wpre, bpre, cb)
    return tok4.reshape(n), zn


def _make_gather(d, n):
    info = plsc.get_sparse_core_info()
    nw = info.num_cores * info.num_subcores
    b_per_w = n // nw
    ch = 128
    nch = b_per_w // ch
    mesh = plsc.VectorSubcoreMesh(core_axis_name="c", subcore_axis_name="s")

    @functools.partial(
        pl.kernel, mesh=mesh,
        out_type=jax.ShapeDtypeStruct((n, d), jnp.float32),
        scratch_types=[
            pltpu.VMEM((b_per_w,), jnp.int32),
            pltpu.VMEM((2, ch, d), jnp.float32),
            pltpu.SemaphoreType.DMA,
            pltpu.SemaphoreType.DMA,
        ],
    )
    def gk(cb_hbm, idx_hbm, q_hbm, idx_v, bufs_v, sem_g, sem_o):
        wid = lax.axis_index("s") * info.num_cores + lax.axis_index("c")
        base = wid * b_per_w

        def gather(j):
            return pltpu.async_copy(cb_hbm.at[idx_v.at[pl.ds(j * ch, ch)]],
                                    bufs_v.at[j % 2], sem_g)

        def put(j):
            return pltpu.async_copy(bufs_v.at[j % 2],
                                    q_hbm.at[pl.ds(base + j * ch, ch)],
                                    sem_o)

        pltpu.sync_copy(idx_hbm.at[pl.ds(base, b_per_w)], idx_v)
        g = [None] * nch
        o = [None] * nch
        g[0] = gather(0)
        g[1] = gather(1)
        for j in range(nch):
            if j >= 1:
                o[j - 1].wait()
                if j + 1 < nch:
                    g[j + 1] = gather(j + 1)
            g[j].wait()
            o[j] = put(j)
        o[nch - 1].wait()

    return gk


def _out_body(q_ref, zn_ref, wpost_ref, bpost_ref, out_ref, loss_ref):
    i = pl.program_id(0)
    j = pl.program_id(1)
    e = zn_ref.shape[1]
    q = q_ref[...][:, :e]
    res = lax.dot_general(q, wpost_ref[...], _CONTRACT_MINOR,
                          preferred_element_type=jnp.float32) + bpost_ref[...]
    out_ref[...] = res.reshape(out_ref.shape)
    dd = zn_ref[...] - q

    @pl.when(jnp.logical_and(i == 0, j == 0))
    def _():
        loss_ref[...] = jnp.zeros(loss_ref.shape, loss_ref.dtype)

    loss_ref[...] = loss_ref[...] + jnp.sum(dd * dd)


def _project_out(q, zn, wpost, bpost, b, t, k, tb):
    n, ep = q.shape
    e = zn.shape[1]
    d = wpost.shape[0]
    r = tb * k
    nt = t // tb
    out, loss_sum = pl.pallas_call(
        _out_body,
        grid=(b, nt),
        in_specs=[
            pl.BlockSpec((r, ep), lambda i, j: (i * nt + j, 0)),
            pl.BlockSpec((r, e), lambda i, j: (i * nt + j, 0)),
            pl.BlockSpec(wpost.shape, lambda i, j: (0, 0)),
            pl.BlockSpec((1, d), lambda i, j: (0, 0)),
        ],
        out_specs=[
            pl.BlockSpec((1, tb, k, d), lambda i, j: (i, j, 0, 0)),
            pl.BlockSpec((1, 1), lambda i, j: (0, 0)),
        ],
        out_shape=[
            jax.ShapeDtypeStruct((b, t, k, d), jnp.float32),
            jax.ShapeDtypeStruct((1, 1), jnp.float32),
        ],
    )(q, zn, wpost, bpost)
    return out, loss_sum


def kernel(z, W_pre, b_pre, codebook, W_post, b_post):
    B, T, K, D = z.shape
    C, E = codebook.shape
    N = B * T * K
    TB = 288

    tokens, zn = _assign(z, W_pre, b_pre.reshape(1, E), codebook, tb=TB)

    cb_pad = jnp.pad(codebook, ((0, 0), (0, _LANES - E)))
    gk = _make_gather(_LANES, N)
    q = gk(cb_pad, tokens)

    out, loss_sum = _project_out(q, zn, W_post, b_post.reshape(1, D),
                                 B, T, K, tb=576)
    commitment_loss = loss_sum[0, 0] * (0.02 / (N * E))
    return (out, tokens.reshape(B, T, K), commitment_loss)


# speedup vs baseline: 1.9361x; 1.0115x over previous
import functools

import jax
import jax.numpy as jnp
from jax import lax
from jax.experimental import pallas as pl
from jax.experimental.pallas import tpu as pltpu
from jax.experimental.pallas import tpu_sc as plsc

_LANES = 128
_CONTRACT_MINOR = (((1,), (1,)), ((), ()))


def _assign_body(z_ref, wpre_ref, bpre_ref, cb_ref, tok_ref, zn_ref):
    zb4 = z_ref[...]
    r = zb4.shape[1] * zb4.shape[2]
    zb = zb4.reshape(r, zb4.shape[3])
    zp = lax.dot_general(zb, wpre_ref[...], _CONTRACT_MINOR,
                         preferred_element_type=jnp.float32) + bpre_ref[...]
    norm = jnp.sqrt(jnp.sum(zp * zp, axis=1, keepdims=True))
    zn = zp / jnp.maximum(norm, 1e-12)
    zn_ref[...] = zn
    c = cb_ref.shape[0]
    chunk = 2048 if c % 2048 == 0 else c
    m_run = None
    g_run = jnp.zeros((r, _LANES), jnp.int32)
    for q2 in range(c // chunk):
        sq = lax.dot_general(zn, cb_ref[pl.ds(q2 * chunk, chunk), :],
                             _CONTRACT_MINOR,
                             preferred_element_type=jnp.float32)
        for g in range(chunk // _LANES):
            sg = sq[:, g * _LANES:(g + 1) * _LANES]
            gg = q2 * (chunk // _LANES) + g
            if m_run is None:
                m_run = sg
                continue
            gt = sg > m_run
            m_run = jnp.where(gt, sg, m_run)
            g_run = jnp.where(gt, gg, g_run)

    lane = lax.broadcasted_iota(jnp.int32, (r, _LANES), 1)
    full_idx = g_run * _LANES + lane
    maxv = jnp.max(m_run, axis=1, keepdims=True)
    eq = m_run == maxv
    tok = jnp.min(jnp.where(eq, full_idx, jnp.int32(0x7FFFFFFF)), axis=1)
    tok_ref[...] = tok.reshape(tok_ref.shape)


def _assign(z4, wpre, bpre, cb, tb):
    b, t, k, d = z4.shape
    e = wpre.shape[0]
    n = b * t * k
    r = tb * k
    nt = t // tb
    tok4, zn = pl.pallas_call(
        _assign_body,
        grid=(b, nt),
        in_specs=[
            pl.BlockSpec((1, tb, k, d), lambda i, j: (i, j, 0, 0)),
            pl.BlockSpec(wpre.shape, lambda i, j: (0, 0)),
            pl.BlockSpec((1, e), lambda i, j: (0, 0)),
            pl.BlockSpec(cb.shape, lambda i, j: (0, 0)),
        ],
        out_specs=[
            pl.BlockSpec((1, 1, 1, r), lambda i, j: (i, j, 0, 0)),
            pl.BlockSpec((r, e), lambda i, j: (i * nt + j, 0)),
        ],
        out_shape=[
            jax.ShapeDtypeStruct((b, nt, 1, r), jnp.int32),
            jax.ShapeDtypeStruct((n, e), jnp.float32),
        ],
    )(z4, wpre, bpre, cb)
    return tok4.reshape(n), zn


def _make_gather(d, n):
    info = plsc.get_sparse_core_info()
    nw = info.num_cores * info.num_subcores
    b_per_w = n // nw
    ch = 128
    nch = b_per_w // ch
    mesh = plsc.VectorSubcoreMesh(core_axis_name="c", subcore_axis_name="s")

    @functools.partial(
        pl.kernel, mesh=mesh,
        out_type=jax.ShapeDtypeStruct((n, d), jnp.float32),
        scratch_types=[
            pltpu.VMEM((b_per_w,), jnp.int32),
            pltpu.VMEM((2, ch, d), jnp.float32),
            pltpu.SemaphoreType.DMA,
            pltpu.SemaphoreType.DMA,
        ],
    )
    def gk(cb_hbm, idx_hbm, q_hbm, idx_v, bufs_v, sem_g, sem_o):
        wid = lax.axis_index("s") * info.num_cores + lax.axis_index("c")
        base = wid * b_per_w

        def gather(j):
            return pltpu.async_copy(cb_hbm.at[idx_v.at[pl.ds(j * ch, ch)]],
                                    bufs_v.at[j % 2], sem_g)

        def put(j):
            return pltpu.async_copy(bufs_v.at[j % 2],
                                    q_hbm.at[pl.ds(base + j * ch, ch)],
                                    sem_o)

        pltpu.sync_copy(idx_hbm.at[pl.ds(base, b_per_w)], idx_v)
        g = [None] * nch
        o = [None] * nch
        g[0] = gather(0)
        g[1] = gather(1)
        for j in range(nch):
            if j >= 1:
                o[j - 1].wait()
                if j + 1 < nch:
                    g[j + 1] = gather(j + 1)
            g[j].wait()
            o[j] = put(j)
        o[nch - 1].wait()

    return gk


def _out_body(q_ref, zn_ref, wpost_ref, bpost_ref, out_ref, loss_ref):
    i = pl.program_id(0)
    j = pl.program_id(1)
    e = zn_ref.shape[1]
    q = q_ref[...][:, :e]
    res = lax.dot_general(q, wpost_ref[...], _CONTRACT_MINOR,
                          preferred_element_type=jnp.float32) + bpost_ref[...]
    out_ref[...] = res.reshape(out_ref.shape)
    dd = zn_ref[...] - q

    @pl.when(jnp.logical_and(i == 0, j == 0))
    def _():
        loss_ref[...] = jnp.zeros(loss_ref.shape, loss_ref.dtype)

    loss_ref[...] = loss_ref[...] + jnp.sum(dd * dd)


def _project_out(q, zn, wpost, bpost, b, t, k, tb):
    n, ep = q.shape
    e = zn.shape[1]
    d = wpost.shape[0]
    r = tb * k
    nt = t // tb
    out, loss_sum = pl.pallas_call(
        _out_body,
        grid=(b, nt),
        in_specs=[
            pl.BlockSpec((r, ep), lambda i, j: (i * nt + j, 0)),
            pl.BlockSpec((r, e), lambda i, j: (i * nt + j, 0)),
            pl.BlockSpec(wpost.shape, lambda i, j: (0, 0)),
            pl.BlockSpec((1, d), lambda i, j: (0, 0)),
        ],
        out_specs=[
            pl.BlockSpec((1, tb, k, d), lambda i, j: (i, j, 0, 0)),
            pl.BlockSpec((1, 1), lambda i, j: (0, 0)),
        ],
        out_shape=[
            jax.ShapeDtypeStruct((b, t, k, d), jnp.float32),
            jax.ShapeDtypeStruct((1, 1), jnp.float32),
        ],
    )(q, zn, wpost, bpost)
    return out, loss_sum


def kernel(z, W_pre, b_pre, codebook, W_post, b_post):
    B, T, K, D = z.shape
    C, E = codebook.shape
    N = B * T * K
    TB = 576

    tokens, zn = _assign(z, W_pre, b_pre.reshape(1, E), codebook, tb=TB)

    cb_pad = jnp.pad(codebook, ((0, 0), (0, _LANES - E)))
    gk = _make_gather(_LANES, N)
    q = gk(cb_pad, tokens)

    out, loss_sum = _project_out(q, zn, W_post, b_post.reshape(1, D),
                                 B, T, K, tb=576)
    commitment_loss = loss_sum[0, 0] * (0.02 / (N * E))
    return (out, tokens.reshape(B, T, K), commitment_loss)
